# baseline (speedup 1.0000x reference)
import functools

import jax
import jax.numpy as jnp
from jax import lax
from jax.experimental import pallas as pl
from jax.experimental.pallas import tpu as pltpu
from jax.experimental.pallas import tpu_sc as plsc

N = 100000
HIDDEN = 128
NUM_CORES = 2
NUM_SUBCORES = 16
NW = NUM_CORES * NUM_SUBCORES
BLK = 320
STEPS = 10
NBUF = 3
SPAN = BLK * STEPS


def _sc_lookup(idx, table):
  mesh = plsc.VectorSubcoreMesh(core_axis_name="c", subcore_axis_name="s")

  @functools.partial(
      pl.kernel,
      mesh=mesh,
      out_type=jax.ShapeDtypeStruct((N, HIDDEN), jnp.float32),
      scratch_types=[
          pltpu.VMEM_SHARED((20, HIDDEN), jnp.float32),
          pltpu.VMEM((SPAN,), jnp.int32),
          pltpu.VMEM((NBUF, BLK, HIDDEN), jnp.float32),
          pltpu.SemaphoreType.DMA,
          pltpu.SemaphoreType.DMA,
          pltpu.SemaphoreType.DMA,
          pltpu.SemaphoreType.DMA,
          pltpu.SemaphoreType.DMA,
          pltpu.SemaphoreType.DMA,
          pltpu.SemaphoreType.DMA,
      ],
  )
  def k(idx_hbm, table_hbm, out_hbm, table_v, idx_v, rows_v,
        isem, gsem0, gsem1, gsem2, ssem0, ssem1, ssem2):
    wid = lax.axis_index("s") * NUM_CORES + lax.axis_index("c")
    gsems = (gsem0, gsem1, gsem2)
    ssems = (ssem0, ssem1, ssem2)

    start = jnp.minimum((wid * (N // NW) // 8) * 8, N - SPAN)

    idx_h = pltpu.async_copy(idx_hbm.at[pl.ds(start, SPAN)], idx_v, isem)

    @pl.when(lax.axis_index("s") == 0)
    def _():
      pltpu.sync_copy(table_hbm, table_v)

    plsc.subcore_barrier()
    idx_h.wait()

    store_h = [None] * NBUF
    gath_h = [None] * NBUF
    for step in range(STEPS + 1):
      if step < STEPS:
        buf = step % NBUF
        if store_h[buf] is not None:
          store_h[buf].wait()
        gath_h[buf] = pltpu.async_copy(
            table_v.at[idx_v.at[pl.ds(step * BLK, BLK)]],
            rows_v.at[buf],
            gsems[buf])
      if step >= 1:
        pbuf = (step - 1) % NBUF
        gath_h[pbuf].wait()
        store_h[pbuf] = pltpu.async_copy(
            rows_v.at[pbuf],
            out_hbm.at[pl.ds(start + (step - 1) * BLK, BLK)],
            ssems[pbuf])

    for h in store_h:
      if h is not None:
        h.wait()

  return k(idx, table)


def kernel(node_degree, degree_emb):
  return _sc_lookup(node_degree.astype(jnp.int32), degree_emb)


# speedup vs baseline: 1.0506x; 1.0506x over previous
import functools

import jax
import jax.numpy as jnp
from jax import lax
from jax.experimental import pallas as pl
from jax.experimental.pallas import tpu as pltpu
from jax.experimental.pallas import tpu_sc as plsc

N = 100000
HIDDEN = 128
NUM_CORES = 2
NUM_SUBCORES = 16
NW = NUM_CORES * NUM_SUBCORES
BLK = 448
STEPS = 7
SPAN = BLK * STEPS


def _sc_lookup(idx, table):
  mesh = plsc.VectorSubcoreMesh(core_axis_name="c", subcore_axis_name="s")

  @functools.partial(
      pl.kernel,
      mesh=mesh,
      out_type=jax.ShapeDtypeStruct((N, HIDDEN), jnp.float32),
      scratch_types=[
          pltpu.VMEM_SHARED((20, HIDDEN), jnp.float32),
          pltpu.VMEM((SPAN,), jnp.int32),
          pltpu.VMEM((2, BLK, HIDDEN), jnp.float32),
          pltpu.SemaphoreType.DMA,
          pltpu.SemaphoreType.DMA,
          pltpu.SemaphoreType.DMA,
          pltpu.SemaphoreType.DMA,
          pltpu.SemaphoreType.DMA,
      ],
  )
  def k(idx_hbm, table_hbm, out_hbm, table_v, idx_v, rows_v,
        isem, gsem0, gsem1, ssem0, ssem1):
    wid = lax.axis_index("s") * NUM_CORES + lax.axis_index("c")
    gsems = (gsem0, gsem1)
    ssems = (ssem0, ssem1)

    start = jnp.minimum((wid * (N // NW) // 8) * 8, N - SPAN)

    idx_h = pltpu.async_copy(idx_hbm.at[pl.ds(start, SPAN)], idx_v, isem)

    @pl.when(lax.axis_index("s") == 0)
    def _():
      pltpu.sync_copy(table_hbm, table_v)

    plsc.subcore_barrier()
    idx_h.wait()

    store_h = [None, None]
    for step in range(STEPS):
      buf = step % 2
      base = start + step * BLK
      if store_h[buf] is not None:
        store_h[buf].wait()
      pltpu.async_copy(
          table_v.at[idx_v.at[pl.ds(step * BLK, BLK)]],
          rows_v.at[buf],
          gsems[buf]).wait()
      store_h[buf] = pltpu.async_copy(
          rows_v.at[buf], out_hbm.at[pl.ds(base, BLK)], ssems[buf])

    store_h[0].wait()
    store_h[1].wait()

  return k(idx, table)


def kernel(node_degree, degree_emb):
  return _sc_lookup(node_degree.astype(jnp.int32), degree_emb)
